# Initial kernel scaffold; baseline (speedup 1.0000x reference)
import jax
import jax.numpy as jnp
from jax.experimental import pallas as pl
from jax.experimental.pallas import tpu as pltpu

_N, _D = 2048, 768
_H, _DH = 12, 64
_C = 500
_CP = 512
_K = 64
_DEPTH = 8
_BLK = 256
_NBLK = _N // _BLK

_pc = pl.pallas_call


def _centroid_body(p_ref, c_ref):
    p = p_ref[...]
    c = ((((p[:, 0, :] + p[:, 1, :]) + p[:, 2, :]) + p[:, 3, :]) + p[:, 4, :]) / 5.0
    c_ref[...] = c


def _prep_body(x_ref, w_ref, b_ref, c_ref, pos_ref, gw_ref, q_ref, k_ref, v_ref, m_ref):
    x = x_ref[...]
    qkv = jax.lax.dot_general(x, w_ref[...], (((1,), (1,)), ((), ())),
                              preferred_element_type=jnp.float32)
    qkv = qkv + b_ref[...]
    q_ref[...] = qkv[:, :_D]
    k_ref[...] = qkv[:, _D:2 * _D]
    v_ref[...] = qkv[:, 2 * _D:]
    nrm = jnp.sqrt(jnp.sum(x * x, axis=1, keepdims=True))
    fn = x / jnp.maximum(nrm, 1e-12)
    sims = jax.lax.dot_general(fn, c_ref[...], (((1,), (1,)), ((), ())),
                               preferred_element_type=jnp.float32)
    lane = jax.lax.broadcasted_iota(jnp.int32, (_BLK, _CP), 1)
    sims = jnp.where(lane < _C, sims, -1e30)
    nearest = jnp.max(sims, axis=1, keepdims=True)
    gd = 1.0 - nearest
    gw = gw_ref[0, 0]
    pos = pos_ref[0]
    xc = jnp.clip(pos * (1.0 - gw) + gd * gw, 1e-6, 1.0 - 1e-6)
    m = jnp.zeros(xc.shape, jnp.int32)
    for t in range(_DEPTH):
        xs = xc * 3.0
        digit = jnp.floor(xs)
        xc = xs - digit
        m = m + jnp.where(digit == 2.0, jnp.int32(1 << (_DEPTH - 1 - t)), 0)
    m_ref[0] = m


def _attn_body(q_ref, k_ref, v_ref, mr_ref, mc_ref, wo_ref, bo_ref, o_ref):
    mi = mr_ref[0]
    mj = mc_ref[...]
    dist = jnp.abs(mi - mj)
    col = jax.lax.broadcasted_iota(jnp.int32, (_BLK, _N), 1)
    key = dist * _N + col

    def bs_step(_, lh):
        lo, hi = lh
        mid = jax.lax.div(lo + hi, 2)
        cnt = jnp.sum(jnp.where(key <= mid, 1.0, 0.0), axis=1, keepdims=True)
        pred = cnt >= float(_K)
        return (jnp.where(pred, lo, mid + 1), jnp.where(pred, mid, hi))

    lo0 = jnp.zeros((_BLK, 1), jnp.int32)
    hi0 = jnp.full((_BLK, 1), _N * 256 - 1, jnp.int32)
    _, thr = jax.lax.fori_loop(0, 19, bs_step, (lo0, hi0))
    mask = key <= thr

    scale = 0.125
    q = q_ref[...]
    k = k_ref[...]
    v = v_ref[...]
    outs = []
    for h in range(_H):
        qh = q[:, h * _DH:(h + 1) * _DH]
        kh = k[:, h * _DH:(h + 1) * _DH]
        vh = v[:, h * _DH:(h + 1) * _DH]
        s = jax.lax.dot_general(qh, kh, (((1,), (1,)), ((), ())),
                                preferred_element_type=jnp.float32) * scale
        s = jnp.where(mask, s, -1e30)
        pmax = jnp.max(s, axis=1, keepdims=True)
        p = jnp.exp(s - pmax)
        denom = jnp.sum(p, axis=1, keepdims=True)
        attn = p / denom
        outs.append(jax.lax.dot_general(attn, vh, (((1,), (0,)), ((), ())),
                                        preferred_element_type=jnp.float32))
    ob = jnp.concatenate(outs, axis=1)
    res = jax.lax.dot_general(ob, wo_ref[...], (((1,), (1,)), ((), ())),
                              preferred_element_type=jnp.float32) + bo_ref[...]
    o_ref[...] = res


def kernel(x, shared_pentachora, W_qkv, b_qkv, W_out, b_out, geo_w):
    b, n, d = x.shape
    x2 = x.reshape(n, d)

    cent = _pc(_centroid_body,
               out_shape=jax.ShapeDtypeStruct((_C, _D), jnp.float32))(shared_pentachora)
    cpad = jnp.pad(cent, ((0, _CP - _C), (0, 0)))

    pos3 = jnp.linspace(0.0, 1.0, n).reshape(_NBLK, _BLK, 1)
    gw = jax.nn.sigmoid(geo_w).reshape(1, 1)

    grid = (_NBLK,)
    q, k, v, m = _pc(
        _prep_body,
        grid=grid,
        in_specs=[
            pl.BlockSpec((_BLK, _D), lambda i: (i, 0)),
            pl.BlockSpec((3 * _D, _D), lambda i: (0, 0)),
            pl.BlockSpec((1, 3 * _D), lambda i: (0, 0)),
            pl.BlockSpec((_CP, _D), lambda i: (0, 0)),
            pl.BlockSpec((1, _BLK, 1), lambda i: (i, 0, 0)),
            pl.BlockSpec((1, 1), lambda i: (0, 0)),
        ],
        out_specs=[
            pl.BlockSpec((_BLK, _D), lambda i: (i, 0)),
            pl.BlockSpec((_BLK, _D), lambda i: (i, 0)),
            pl.BlockSpec((_BLK, _D), lambda i: (i, 0)),
            pl.BlockSpec((1, _BLK, 1), lambda i: (i, 0, 0)),
        ],
        out_shape=[
            jax.ShapeDtypeStruct((n, d), jnp.float32),
            jax.ShapeDtypeStruct((n, d), jnp.float32),
            jax.ShapeDtypeStruct((n, d), jnp.float32),
            jax.ShapeDtypeStruct((_NBLK, _BLK, 1), jnp.int32),
        ],
    )(x2, W_qkv, b_qkv.reshape(1, 3 * _D), cpad, pos3, gw)

    mc = m.reshape(1, n)

    out = _pc(
        _attn_body,
        grid=grid,
        in_specs=[
            pl.BlockSpec((_BLK, _D), lambda i: (i, 0)),
            pl.BlockSpec((_N, _D), lambda i: (0, 0)),
            pl.BlockSpec((_N, _D), lambda i: (0, 0)),
            pl.BlockSpec((1, _BLK, 1), lambda i: (i, 0, 0)),
            pl.BlockSpec((1, _N), lambda i: (0, 0)),
            pl.BlockSpec((_D, _D), lambda i: (0, 0)),
            pl.BlockSpec((1, _D), lambda i: (0, 0)),
        ],
        out_specs=pl.BlockSpec((_BLK, _D), lambda i: (i, 0)),
        out_shape=jax.ShapeDtypeStruct((n, d), jnp.float32),
    )(q, k, v, m, mc, W_out, b_out.reshape(1, _D))
    return out.reshape(b, n, d)


# speedup vs baseline: 23.1533x; 23.1533x over previous
import jax
import jax.numpy as jnp
from jax.experimental import pallas as pl
from jax.experimental.pallas import tpu as pltpu

_N, _D = 2048, 768
_H, _DH = 12, 64
_C = 500
_CP = 512
_K = 64
_DEPTH = 8
_BLK = 256
_NBLK = _N // _BLK

_pc = pl.pallas_call


def _centroid_body(p_ref, c_ref):
    p = p_ref[...]
    c = ((((p[:, 0, :] + p[:, 1, :]) + p[:, 2, :]) + p[:, 3, :]) + p[:, 4, :]) / 5.0
    c_ref[...] = c


def _prep_body(x_ref, fn_ref, w_ref, b_ref, c_ref, pos_ref, gw_ref,
               q_ref, k_ref, v_ref, m_ref):
    x = x_ref[...]
    qkv = jax.lax.dot_general(x, w_ref[...], (((1,), (1,)), ((), ())),
                              preferred_element_type=jnp.float32)
    qkv = qkv + b_ref[...]
    q_ref[...] = qkv[:, :_D]
    k_ref[...] = qkv[:, _D:2 * _D]
    v_ref[...] = qkv[:, 2 * _D:]
    sims = jax.lax.dot_general(fn_ref[...], c_ref[...], (((1,), (1,)), ((), ())),
                               preferred_element_type=jnp.float32)
    lane = jax.lax.broadcasted_iota(jnp.int32, (_BLK, _CP), 1)
    sims = jnp.where(lane < _C, sims, -1e30)
    nearest = jnp.max(sims, axis=1, keepdims=True)
    gd = 1.0 - nearest
    gw = gw_ref[0, 0]
    pos = pos_ref[0]
    xc = jnp.clip(pos * (1.0 - gw) + gd * gw, 1e-6, 1.0 - 1e-6)
    m = jnp.zeros(xc.shape, jnp.int32)
    for t in range(_DEPTH):
        xs = xc * 3.0
        digit = jnp.floor(xs)
        xc = xs - digit
        m = m + jnp.where(digit == 2.0, jnp.int32(1 << (_DEPTH - 1 - t)), 0)
    m_ref[0] = m


def _attn_body(q_ref, k_ref, v_ref, mr_ref, mc_ref, wo_ref, bo_ref, o_ref):
    mi = mr_ref[0]
    mj = mc_ref[...]
    dist = jnp.abs(mi - mj)
    col = jax.lax.broadcasted_iota(jnp.int32, (_BLK, _N), 1)
    key = dist * _N + col

    def bs_step(_, lh):
        lo, hi = lh
        mid = jax.lax.div(lo + hi, 2)
        cnt = jnp.sum(jnp.where(key <= mid, 1.0, 0.0), axis=1, keepdims=True)
        pred = cnt >= float(_K)
        return (jnp.where(pred, lo, mid + 1), jnp.where(pred, mid, hi))

    lo0 = jnp.zeros((_BLK, 1), jnp.int32)
    hi0 = jnp.full((_BLK, 1), _N * 256 - 1, jnp.int32)
    _, thr = jax.lax.fori_loop(0, 19, bs_step, (lo0, hi0))
    mask = key <= thr

    scale = 0.125
    q = q_ref[...]
    k = k_ref[...]
    v = v_ref[...]
    outs = []
    for h in range(_H):
        qh = q[:, h * _DH:(h + 1) * _DH]
        kh = k[:, h * _DH:(h + 1) * _DH]
        vh = v[:, h * _DH:(h + 1) * _DH]
        s = jax.lax.dot_general(qh, kh, (((1,), (1,)), ((), ())),
                                preferred_element_type=jnp.float32) * scale
        s = jnp.where(mask, s, -1e30)
        pmax = jnp.max(s, axis=1, keepdims=True)
        p = jnp.exp(s - pmax)
        denom = jnp.sum(p, axis=1, keepdims=True)
        attn = p / denom
        outs.append(jax.lax.dot_general(attn, vh, (((1,), (0,)), ((), ())),
                                        preferred_element_type=jnp.float32))
    ob = jnp.concatenate(outs, axis=1)
    res = jax.lax.dot_general(ob, wo_ref[...], (((1,), (1,)), ((), ())),
                              preferred_element_type=jnp.float32) + bo_ref[...]
    o_ref[...] = res


def kernel(x, shared_pentachora, W_qkv, b_qkv, W_out, b_out, geo_w):
    b, n, d = x.shape
    x2 = x.reshape(n, d)

    cent = _pc(_centroid_body,
               out_shape=jax.ShapeDtypeStruct((_C, _D), jnp.float32))(shared_pentachora)
    cpad = jnp.pad(cent, ((0, _CP - _C), (0, 0)))

    pos3 = jnp.linspace(0.0, 1.0, n).reshape(_NBLK, _BLK, 1)
    gw = jax.nn.sigmoid(geo_w).reshape(1, 1)
    fn = (x / jnp.maximum(jnp.linalg.norm(x, axis=-1, keepdims=True),
                          1e-12)).reshape(n, d)

    grid = (_NBLK,)
    q, k, v, m = _pc(
        _prep_body,
        grid=grid,
        in_specs=[
            pl.BlockSpec((_BLK, _D), lambda i: (i, 0)),
            pl.BlockSpec((_BLK, _D), lambda i: (i, 0)),
            pl.BlockSpec((3 * _D, _D), lambda i: (0, 0)),
            pl.BlockSpec((1, 3 * _D), lambda i: (0, 0)),
            pl.BlockSpec((_CP, _D), lambda i: (0, 0)),
            pl.BlockSpec((1, _BLK, 1), lambda i: (i, 0, 0)),
            pl.BlockSpec((1, 1), lambda i: (0, 0)),
        ],
        out_specs=[
            pl.BlockSpec((_BLK, _D), lambda i: (i, 0)),
            pl.BlockSpec((_BLK, _D), lambda i: (i, 0)),
            pl.BlockSpec((_BLK, _D), lambda i: (i, 0)),
            pl.BlockSpec((1, _BLK, 1), lambda i: (i, 0, 0)),
        ],
        out_shape=[
            jax.ShapeDtypeStruct((n, d), jnp.float32),
            jax.ShapeDtypeStruct((n, d), jnp.float32),
            jax.ShapeDtypeStruct((n, d), jnp.float32),
            jax.ShapeDtypeStruct((_NBLK, _BLK, 1), jnp.int32),
        ],
    )(x2, fn, W_qkv, b_qkv.reshape(1, 3 * _D), cpad, pos3, gw)

    mc = m.reshape(1, n)

    out = _pc(
        _attn_body,
        grid=grid,
        in_specs=[
            pl.BlockSpec((_BLK, _D), lambda i: (i, 0)),
            pl.BlockSpec((_N, _D), lambda i: (0, 0)),
            pl.BlockSpec((_N, _D), lambda i: (0, 0)),
            pl.BlockSpec((1, _BLK, 1), lambda i: (i, 0, 0)),
            pl.BlockSpec((1, _N), lambda i: (0, 0)),
            pl.BlockSpec((_D, _D), lambda i: (0, 0)),
            pl.BlockSpec((1, _D), lambda i: (0, 0)),
        ],
        out_specs=pl.BlockSpec((_BLK, _D), lambda i: (i, 0)),
        out_shape=jax.ShapeDtypeStruct((n, d), jnp.float32),
    )(q, k, v, m, mc, W_out, b_out.reshape(1, _D))
    return out.reshape(b, n, d)


# speedup vs baseline: 32.5422x; 1.4055x over previous
import jax
import jax.numpy as jnp
from jax.experimental import pallas as pl
from jax.experimental.pallas import tpu as pltpu

_N, _D = 2048, 768
_H, _DH = 12, 64
_C = 500
_CP = 512
_K = 64
_DEPTH = 8
_BLK = 256
_NBLK = _N // _BLK
_NV = 256

_pc = pl.pallas_call


def _centroid_body(p_ref, c_ref):
    p = p_ref[...]
    c = ((((p[:, 0, :] + p[:, 1, :]) + p[:, 2, :]) + p[:, 3, :]) + p[:, 4, :]) / 5.0
    c_ref[:_C, :] = c
    c_ref[_C:, :] = jnp.zeros((_CP - _C, _D), jnp.float32)


def _prep_body(x_ref, fn_ref, w_ref, b_ref, c_ref, pos_ref, gw_ref,
               q_ref, k_ref, v_ref, m_ref):
    x = x_ref[...]
    qkv = jax.lax.dot_general(x, w_ref[...], (((1,), (1,)), ((), ())),
                              preferred_element_type=jnp.float32)
    qkv = qkv + b_ref[...]
    for h in range(_H):
        q_ref[h] = qkv[:, h * _DH:(h + 1) * _DH] * 0.125
        k_ref[h] = qkv[:, _D + h * _DH:_D + (h + 1) * _DH]
        v_ref[h] = qkv[:, 2 * _D + h * _DH:2 * _D + (h + 1) * _DH]
    sims = jax.lax.dot_general(fn_ref[...], c_ref[...], (((1,), (1,)), ((), ())),
                               preferred_element_type=jnp.float32)
    lane = jax.lax.broadcasted_iota(jnp.int32, (_BLK, _CP), 1)
    sims = jnp.where(lane < _C, sims, -1e30)
    nearest = jnp.max(sims, axis=1, keepdims=True)
    gd = 1.0 - nearest
    gw = gw_ref[0, 0]
    pos = pos_ref[0]
    xc = jnp.clip(pos * (1.0 - gw) + gd * gw, 1e-6, 1.0 - 1e-6)
    m = jnp.zeros(xc.shape, jnp.int32)
    for t in range(_DEPTH):
        xs = xc * 3.0
        digit = jnp.floor(xs)
        xc = xs - digit
        m = m + jnp.where(digit == 2.0, jnp.int32(1 << (_DEPTH - 1 - t)), 0)
    m_ref[0] = m


def _attn_body(q_ref, k_ref, v_ref, mr_ref, mc_ref, mcol_ref, wo_ref, bo_ref,
               o_ref, t_ref):
    mj = mc_ref[...].astype(jnp.float32)

    @pl.when(pl.program_id(0) == 0)
    def _build_table():
        val = jax.lax.broadcasted_iota(jnp.int32, (1, _NV), 1).astype(jnp.float32)
        mjc = mcol_ref[...].astype(jnp.float32)
        colv = jax.lax.broadcasted_iota(jnp.int32, (_N, 1), 0).astype(jnp.float32)
        keyv = jnp.abs(mjc - val) * float(_N) + colv

        def bs_step(_, lh):
            lo, hi = lh
            mid = jnp.floor((lo + hi) * 0.5)
            cnt = jnp.sum(jnp.where(keyv <= mid, 1.0, 0.0), axis=0,
                          keepdims=True)
            pred = cnt >= float(_K)
            return (jnp.where(pred, lo, mid + 1.0), jnp.where(pred, mid, hi))

        lo0 = jnp.zeros((1, _NV), jnp.float32)
        hi0 = jnp.full((1, _NV), float(_N * 256 - 1), jnp.float32)
        _, thr = jax.lax.fori_loop(0, 19, bs_step, (lo0, hi0))
        t_ref[...] = thr

    mi = mr_ref[0]
    onehot = (mi == jax.lax.broadcasted_iota(jnp.int32, (_BLK, _NV), 1))
    thr = jnp.sum(jnp.where(onehot, t_ref[...], 0.0), axis=1,
                  keepdims=True)
    mi_f = mi.astype(jnp.float32)
    col = jax.lax.broadcasted_iota(jnp.int32, (_BLK, _N), 1).astype(jnp.float32)
    key = jnp.abs(mi_f - mj) * float(_N) + col
    mask = key <= thr

    outs = []
    for h in range(_H):
        s = jax.lax.dot_general(q_ref[h], k_ref[h], (((1,), (1,)), ((), ())),
                                preferred_element_type=jnp.float32)
        s = jnp.where(mask, s, -1e30)
        pmax = jnp.max(s, axis=1, keepdims=True)
        p = jnp.exp(s - pmax)
        denom = jnp.sum(p, axis=1, keepdims=True)
        oh = jax.lax.dot_general(p, v_ref[h], (((1,), (0,)), ((), ())),
                                 preferred_element_type=jnp.float32)
        outs.append(oh * (1.0 / denom))
    ob = jnp.concatenate(outs, axis=1)
    res = jax.lax.dot_general(ob, wo_ref[...], (((1,), (1,)), ((), ())),
                              preferred_element_type=jnp.float32) + bo_ref[...]
    o_ref[...] = res


def kernel(x, shared_pentachora, W_qkv, b_qkv, W_out, b_out, geo_w):
    b, n, d = x.shape
    x2 = x.reshape(n, d)

    cpad = _pc(_centroid_body,
               out_shape=jax.ShapeDtypeStruct((_CP, _D), jnp.float32))(
                   shared_pentachora)

    pos3 = jnp.linspace(0.0, 1.0, n).reshape(_NBLK, _BLK, 1)
    gw = jax.nn.sigmoid(geo_w).reshape(1, 1)
    fn = (x / jnp.maximum(jnp.linalg.norm(x, axis=-1, keepdims=True),
                          1e-12)).reshape(n, d)

    grid = (_NBLK,)
    hspec = pl.BlockSpec((_H, _BLK, _DH), lambda i: (0, i, 0))
    hshape = jax.ShapeDtypeStruct((_H, n, _DH), jnp.float32)
    q, k, v, m = _pc(
        _prep_body,
        grid=grid,
        in_specs=[
            pl.BlockSpec((_BLK, _D), lambda i: (i, 0)),
            pl.BlockSpec((_BLK, _D), lambda i: (i, 0)),
            pl.BlockSpec((3 * _D, _D), lambda i: (0, 0)),
            pl.BlockSpec((1, 3 * _D), lambda i: (0, 0)),
            pl.BlockSpec((_CP, _D), lambda i: (0, 0)),
            pl.BlockSpec((1, _BLK, 1), lambda i: (i, 0, 0)),
            pl.BlockSpec((1, 1), lambda i: (0, 0)),
        ],
        out_specs=[hspec, hspec, hspec,
                   pl.BlockSpec((1, _BLK, 1), lambda i: (i, 0, 0))],
        out_shape=[hshape, hshape, hshape,
                   jax.ShapeDtypeStruct((_NBLK, _BLK, 1), jnp.int32)],
    )(x2, fn, W_qkv, b_qkv.reshape(1, 3 * _D), cpad, pos3, gw)

    mc = m.reshape(1, n)

    out = _pc(
        _attn_body,
        grid=grid,
        in_specs=[
            pl.BlockSpec((_H, _BLK, _DH), lambda i: (0, i, 0)),
            pl.BlockSpec((_H, _N, _DH), lambda i: (0, 0, 0)),
            pl.BlockSpec((_H, _N, _DH), lambda i: (0, 0, 0)),
            pl.BlockSpec((1, _BLK, 1), lambda i: (i, 0, 0)),
            pl.BlockSpec((1, _N), lambda i: (0, 0)),
            pl.BlockSpec((_N, 1), lambda i: (0, 0)),
            pl.BlockSpec((_D, _D), lambda i: (0, 0)),
            pl.BlockSpec((1, _D), lambda i: (0, 0)),
        ],
        out_specs=pl.BlockSpec((_BLK, _D), lambda i: (i, 0)),
        out_shape=jax.ShapeDtypeStruct((n, d), jnp.float32),
        scratch_shapes=[pltpu.VMEM((1, _NV), jnp.float32)],
    )(q, k, v, m, mc, m.reshape(n, 1), W_out, b_out.reshape(1, _D))
    return out.reshape(b, n, d)


# speedup vs baseline: 42.3117x; 1.3002x over previous
import jax
import jax.numpy as jnp
from jax.experimental import pallas as pl
from jax.experimental.pallas import tpu as pltpu

_N, _D = 2048, 768
_H, _DH = 12, 64
_C = 500
_CP = 512
_K = 64
_DEPTH = 8
_BLK = 256
_NBLK = _N // _BLK
_NV = 256

_pc = pl.pallas_call


def _centroid_body(p_ref, c_ref):
    p = p_ref[...]
    c = ((((p[:, 0, :] + p[:, 1, :]) + p[:, 2, :]) + p[:, 3, :]) + p[:, 4, :]) / 5.0
    c_ref[:_C, :] = c
    c_ref[_C:, :] = jnp.zeros((_CP - _C, _D), jnp.float32)


def _fused_body(x_ref, fn_ref, w_ref, b_ref, c_ref, pos_ref, gw_ref, eye_ref,
                wo_ref, bo_ref, o_ref,
                q_scr, k_scr, v_scr, mcol_scr, mrow_scr, t_scr):
    i = pl.program_id(0)

    @pl.when(i < _NBLK)
    def _prep():
        base = i * _BLK
        x = x_ref[...]
        qkv = jax.lax.dot_general(x, w_ref[...], (((1,), (1,)), ((), ())),
                                  preferred_element_type=jnp.float32)
        qkv = qkv + b_ref[...]
        lane64 = jax.lax.broadcasted_iota(jnp.int32, (_BLK, _DH), 1)
        onescol = jnp.where(lane64 == 0, 1.0, 0.0)
        for h in range(_H):
            q_scr[h, pl.ds(base, _BLK), :] = (
                qkv[:, h * _DH:(h + 1) * _DH] * 0.125).astype(jnp.bfloat16)
            k_scr[h, pl.ds(base, _BLK), :] = (
                qkv[:, _D + h * _DH:_D + (h + 1) * _DH]).astype(jnp.bfloat16)
            v_scr[h, pl.ds(base, _BLK), 0:_DH] = \
                qkv[:, 2 * _D + h * _DH:2 * _D + (h + 1) * _DH]
            v_scr[h, pl.ds(base, _BLK), _DH:2 * _DH] = onescol
        sims = jax.lax.dot_general(fn_ref[...], c_ref[...],
                                   (((1,), (1,)), ((), ())),
                                   preferred_element_type=jnp.float32)
        lane = jax.lax.broadcasted_iota(jnp.int32, (_BLK, _CP), 1)
        sims = jnp.where(lane < _C, sims, -1e30)
        nearest = jnp.max(sims, axis=1, keepdims=True)
        gd = 1.0 - nearest
        gw = gw_ref[0, 0]
        pos = pos_ref[0]
        xc = jnp.clip(pos * (1.0 - gw) + gd * gw, 1e-6, 1.0 - 1e-6)
        m = jnp.zeros(xc.shape, jnp.float32)
        for t in range(_DEPTH):
            xs = xc * 3.0
            digit = jnp.floor(xs)
            xc = xs - digit
            m = m + jnp.where(digit == 2.0, float(1 << (_DEPTH - 1 - t)), 0.0)
        mcol_scr[pl.ds(base, _BLK), :] = m
        mrow_scr[0:1, pl.ds(base, _BLK)] = jax.lax.dot_general(
            m, eye_ref[...], (((0,), (0,)), ((), ())),
            preferred_element_type=jnp.float32)

    @pl.when(i == _NBLK)
    def _build_table():
        val = jax.lax.broadcasted_iota(jnp.int32, (1, _NV), 1).astype(jnp.float32)
        mjc = mcol_scr[...]
        colv = jax.lax.broadcasted_iota(jnp.int32, (_N, 1), 0).astype(jnp.float32)
        keyv = jnp.abs(mjc - val) * float(_N) + colv

        def bs_step(_, lh):
            lo, hi = lh
            mid = jnp.floor((lo + hi) * 0.5)
            cnt = jnp.sum(jnp.where(keyv <= mid, 1.0, 0.0), axis=0,
                          keepdims=True)
            pred = cnt >= float(_K)
            return (jnp.where(pred, lo, mid + 1.0), jnp.where(pred, mid, hi))

        lo0 = jnp.zeros((1, _NV), jnp.float32)
        hi0 = jnp.full((1, _NV), float(_N * 256 - 1), jnp.float32)
        _, thr = jax.lax.fori_loop(0, 19, bs_step, (lo0, hi0))
        t_scr[...] = thr

    @pl.when(i >= _NBLK)
    def _attn():
        base = (i - _NBLK) * _BLK
        mi = mcol_scr[pl.ds(base, _BLK), :]
        mj = mrow_scr[...]
        vals = jax.lax.broadcasted_iota(jnp.int32, (_BLK, _NV), 1).astype(jnp.float32)
        onehot = mi == vals
        thr = jnp.sum(jnp.where(onehot, t_scr[...], 0.0), axis=1,
                      keepdims=True)
        col = jax.lax.broadcasted_iota(jnp.int32, (_BLK, _N), 1).astype(jnp.float32)
        key = jnp.abs(mi - mj) * float(_N) + col
        mask = key <= thr

        outs = []
        for h in range(_H):
            qh = q_scr[h, pl.ds(base, _BLK), :]
            s = jax.lax.dot_general(qh, k_scr[h], (((1,), (1,)), ((), ())),
                                    preferred_element_type=jnp.float32)
            p = jnp.exp(jnp.where(mask, s, -1e30))
            oha = jax.lax.dot_general(p, v_scr[h], (((1,), (0,)), ((), ())),
                                      preferred_element_type=jnp.float32)
            denom = oha[:, _DH:_DH + 1]
            outs.append(oha[:, :_DH] * (1.0 / denom))
        ob = jnp.concatenate(outs, axis=1)
        res = jax.lax.dot_general(ob, wo_ref[...], (((1,), (1,)), ((), ())),
                                  preferred_element_type=jnp.float32) + bo_ref[...]
        o_ref[...] = res


def kernel(x, shared_pentachora, W_qkv, b_qkv, W_out, b_out, geo_w):
    b, n, d = x.shape
    x2 = x.reshape(n, d)

    cpad = _pc(_centroid_body,
               out_shape=jax.ShapeDtypeStruct((_CP, _D), jnp.float32))(
                   shared_pentachora)

    pos3 = jnp.linspace(0.0, 1.0, n).reshape(_NBLK, _BLK, 1)
    gw = jax.nn.sigmoid(geo_w).reshape(1, 1)
    eye = jnp.eye(_BLK, dtype=jnp.float32)
    fn = (x / jnp.maximum(jnp.linalg.norm(x, axis=-1, keepdims=True),
                          1e-12)).reshape(n, d)

    def blk_or0(i):
        return (jnp.where(i < _NBLK, i, 0), 0)

    def blk3_or0(i):
        return (jnp.where(i < _NBLK, i, 0), 0, 0)

    out = _pc(
        _fused_body,
        grid=(2 * _NBLK,),
        in_specs=[
            pl.BlockSpec((_BLK, _D), blk_or0),
            pl.BlockSpec((_BLK, _D), blk_or0),
            pl.BlockSpec((3 * _D, _D), lambda i: (0, 0)),
            pl.BlockSpec((1, 3 * _D), lambda i: (0, 0)),
            pl.BlockSpec((_CP, _D), lambda i: (0, 0)),
            pl.BlockSpec((1, _BLK, 1), blk3_or0),
            pl.BlockSpec((1, 1), lambda i: (0, 0)),
            pl.BlockSpec((_BLK, _BLK), lambda i: (0, 0)),
            pl.BlockSpec((_D, _D), lambda i: (0, 0)),
            pl.BlockSpec((1, _D), lambda i: (0, 0)),
        ],
        out_specs=pl.BlockSpec(
            (_BLK, _D), lambda i: (jnp.where(i >= _NBLK, i - _NBLK, 0), 0)),
        out_shape=jax.ShapeDtypeStruct((n, d), jnp.float32),
        scratch_shapes=[
            pltpu.VMEM((_H, _N, _DH), jnp.bfloat16),
            pltpu.VMEM((_H, _N, _DH), jnp.bfloat16),
            pltpu.VMEM((_H, _N, 2 * _DH), jnp.float32),
            pltpu.VMEM((_N, 1), jnp.float32),
            pltpu.VMEM((1, _N), jnp.float32),
            pltpu.VMEM((1, _NV), jnp.float32),
        ],
    )(x2, fn, W_qkv, b_qkv.reshape(1, 3 * _D), cpad, pos3, gw, eye,
      W_out, b_out.reshape(1, _D))
    return out.reshape(b, n, d)


# speedup vs baseline: 43.5577x; 1.0294x over previous
import jax
import jax.numpy as jnp
from jax.experimental import pallas as pl
from jax.experimental.pallas import tpu as pltpu

_N, _D = 2048, 768
_H, _DH = 12, 64
_C = 500
_CP = 512
_K = 64
_DEPTH = 8
_BLK = 512
_NBLK = _N // _BLK
_NV = 256

_pc = pl.pallas_call


def _centroid_body(p_ref, c_ref):
    p = p_ref[...]
    c = ((((p[:, 0, :] + p[:, 1, :]) + p[:, 2, :]) + p[:, 3, :]) + p[:, 4, :]) / 5.0
    c_ref[:_C, :] = c
    c_ref[_C:, :] = jnp.zeros((_CP - _C, _D), jnp.float32)


def _fused_body(x_ref, fn_ref, w_ref, b_ref, c_ref, pos_ref, gw_ref, eye_ref,
                wo_ref, bo_ref, o_ref,
                q_scr, k_scr, v_scr, mcol_scr, mrow_scr, t_scr):
    i = pl.program_id(0)

    @pl.when(i < _NBLK)
    def _prep():
        base = i * _BLK
        x = x_ref[...]
        qkv = jax.lax.dot_general(x, w_ref[...], (((1,), (1,)), ((), ())),
                                  preferred_element_type=jnp.float32)
        qkv = qkv + b_ref[...]
        lane64 = jax.lax.broadcasted_iota(jnp.int32, (_BLK, _DH), 1)
        onescol = jnp.where(lane64 == 0, 1.0, 0.0)
        for h in range(_H):
            q_scr[h, pl.ds(base, _BLK), :] = (
                qkv[:, h * _DH:(h + 1) * _DH] * 0.125).astype(jnp.bfloat16)
            k_scr[h, pl.ds(base, _BLK), :] = (
                qkv[:, _D + h * _DH:_D + (h + 1) * _DH]).astype(jnp.bfloat16)
            v_scr[h, pl.ds(base, _BLK), 0:_DH] = \
                qkv[:, 2 * _D + h * _DH:2 * _D + (h + 1) * _DH]
            v_scr[h, pl.ds(base, _BLK), _DH:2 * _DH] = onescol
        sims = jax.lax.dot_general(fn_ref[...], c_ref[...],
                                   (((1,), (1,)), ((), ())),
                                   preferred_element_type=jnp.float32)
        lane = jax.lax.broadcasted_iota(jnp.int32, (_BLK, _CP), 1)
        sims = jnp.where(lane < _C, sims, -1e30)
        nearest = jnp.max(sims, axis=1, keepdims=True)
        gd = 1.0 - nearest
        gw = gw_ref[0, 0]
        pos = pos_ref[0]
        xc = jnp.clip(pos * (1.0 - gw) + gd * gw, 1e-6, 1.0 - 1e-6)
        m = jnp.zeros(xc.shape, jnp.float32)
        for t in range(_DEPTH):
            xs = xc * 3.0
            digit = jnp.floor(xs)
            xc = xs - digit
            m = m + jnp.where(digit == 2.0, float(1 << (_DEPTH - 1 - t)), 0.0)
        mcol_scr[pl.ds(base, _BLK), :] = m
        mrow_scr[0:1, pl.ds(base, _BLK)] = jax.lax.dot_general(
            m, eye_ref[...], (((0,), (0,)), ((), ())),
            preferred_element_type=jnp.float32)

    @pl.when(i == _NBLK)
    def _build_table():
        val = jax.lax.broadcasted_iota(jnp.int32, (1, _NV), 1).astype(jnp.float32)
        mjc = mcol_scr[...]
        colv = jax.lax.broadcasted_iota(jnp.int32, (_N, 1), 0).astype(jnp.float32)
        keyv = jnp.abs(mjc - val) * float(_N) + colv

        def bs_step(_, lh):
            lo, hi = lh
            mid = jnp.floor((lo + hi) * 0.5)
            cnt = jnp.sum(jnp.where(keyv <= mid, 1.0, 0.0), axis=0,
                          keepdims=True)
            pred = cnt >= float(_K)
            return (jnp.where(pred, lo, mid + 1.0), jnp.where(pred, mid, hi))

        lo0 = jnp.zeros((1, _NV), jnp.float32)
        hi0 = jnp.full((1, _NV), float(_N * 256 - 1), jnp.float32)
        _, thr = jax.lax.fori_loop(0, 19, bs_step, (lo0, hi0))
        t_scr[...] = thr

    @pl.when(i >= _NBLK)
    def _attn():
        base = (i - _NBLK) * _BLK
        mi = mcol_scr[pl.ds(base, _BLK), :]
        mj = mrow_scr[...]
        vals = jax.lax.broadcasted_iota(jnp.int32, (_BLK, _NV), 1).astype(jnp.float32)
        onehot = mi == vals
        thr = jnp.sum(jnp.where(onehot, t_scr[...], 0.0), axis=1,
                      keepdims=True)
        col = jax.lax.broadcasted_iota(jnp.int32, (_BLK, _N), 1).astype(jnp.float32)
        key = jnp.abs(mi - mj) * float(_N) + col
        mask = key <= thr

        outs = []
        for h in range(_H):
            qh = q_scr[h, pl.ds(base, _BLK), :]
            s = jax.lax.dot_general(qh, k_scr[h], (((1,), (1,)), ((), ())),
                                    preferred_element_type=jnp.float32)
            p = jnp.exp(jnp.where(mask, s, -1e30))
            oha = jax.lax.dot_general(p, v_scr[h], (((1,), (0,)), ((), ())),
                                      preferred_element_type=jnp.float32)
            denom = oha[:, _DH:_DH + 1]
            outs.append(oha[:, :_DH] * (1.0 / denom))
        ob = jnp.concatenate(outs, axis=1)
        res = jax.lax.dot_general(ob, wo_ref[...], (((1,), (1,)), ((), ())),
                                  preferred_element_type=jnp.float32) + bo_ref[...]
        o_ref[...] = res


def kernel(x, shared_pentachora, W_qkv, b_qkv, W_out, b_out, geo_w):
    b, n, d = x.shape
    x2 = x.reshape(n, d)

    cpad = _pc(_centroid_body,
               out_shape=jax.ShapeDtypeStruct((_CP, _D), jnp.float32))(
                   shared_pentachora)

    pos3 = jnp.linspace(0.0, 1.0, n).reshape(_NBLK, _BLK, 1)
    gw = jax.nn.sigmoid(geo_w).reshape(1, 1)
    eye = jnp.eye(_BLK, dtype=jnp.float32)
    fn = (x / jnp.maximum(jnp.linalg.norm(x, axis=-1, keepdims=True),
                          1e-12)).reshape(n, d)

    def blk_or0(i):
        return (jnp.where(i < _NBLK, i, 0), 0)

    def blk3_or0(i):
        return (jnp.where(i < _NBLK, i, 0), 0, 0)

    out = _pc(
        _fused_body,
        grid=(2 * _NBLK,),
        in_specs=[
            pl.BlockSpec((_BLK, _D), blk_or0),
            pl.BlockSpec((_BLK, _D), blk_or0),
            pl.BlockSpec((3 * _D, _D), lambda i: (0, 0)),
            pl.BlockSpec((1, 3 * _D), lambda i: (0, 0)),
            pl.BlockSpec((_CP, _D), lambda i: (0, 0)),
            pl.BlockSpec((1, _BLK, 1), blk3_or0),
            pl.BlockSpec((1, 1), lambda i: (0, 0)),
            pl.BlockSpec((_BLK, _BLK), lambda i: (0, 0)),
            pl.BlockSpec((_D, _D), lambda i: (0, 0)),
            pl.BlockSpec((1, _D), lambda i: (0, 0)),
        ],
        out_specs=pl.BlockSpec(
            (_BLK, _D), lambda i: (jnp.where(i >= _NBLK, i - _NBLK, 0), 0)),
        out_shape=jax.ShapeDtypeStruct((n, d), jnp.float32),
        scratch_shapes=[
            pltpu.VMEM((_H, _N, _DH), jnp.bfloat16),
            pltpu.VMEM((_H, _N, _DH), jnp.bfloat16),
            pltpu.VMEM((_H, _N, 2 * _DH), jnp.float32),
            pltpu.VMEM((_N, 1), jnp.float32),
            pltpu.VMEM((1, _N), jnp.float32),
            pltpu.VMEM((1, _NV), jnp.float32),
        ],
    )(x2, fn, W_qkv, b_qkv.reshape(1, 3 * _D), cpad, pos3, gw, eye,
      W_out, b_out.reshape(1, _D))
    return out.reshape(b, n, d)
